# baseline (speedup 1.0000x reference)
import functools

import jax
import jax.numpy as jnp
from jax import lax
from jax.experimental import pallas as pl
from jax.experimental.pallas import tpu as pltpu
from jax.experimental.pallas import tpu_sc as plsc

N_AUTHORS = 5000
N_PAPERS = 5000
N = N_AUTHORS + N_PAPERS
E = 320000
D = 128
L = 3

NC = 2
NS = 16
NW = NC * NS
EPW = E // NW
CHUNK = 80
NCHUNK = EPW // CHUNK

OBUF_ROWS = 80
NRCHUNK = N // OBUF_ROWS
RITER = -(-NRCHUNK // NS)


def _spmm_body(x_hbm, col_hbm, row_hbm, val_hbm, out_hbm,
               colv, rowv, valv, rows0, rows1, rows2, acc,
               gs0, gs1, gs2, is0, is1, is2):
    cid = lax.axis_index("c")
    sid = lax.axis_index("s")
    wid = cid * NS + sid
    rows = (rows0, rows1, rows2)
    gsem = (gs0, gs1, gs2)
    isem = (is0, is1, is2)
    obuf = rows0.at[pl.ds(0, OBUF_ROWS)]

    zv = jnp.zeros((16,), jnp.float32)

    def zero_row(i, carry):
        for db in range(D // 16):
            rows0[i, pl.ds(db * 16, 16)] = zv
        return carry

    lax.fori_loop(0, OBUF_ROWS, zero_row, 0)
    for kk in range(RITER):
        rc = sid + kk * NS

        @pl.when(rc < NRCHUNK)
        def _():
            pltpu.sync_copy(obuf, acc.at[pl.ds(rc * OBUF_ROWS, OBUF_ROWS)])
    plsc.subcore_barrier()

    e0 = wid * EPW

    def start_idx(j, ib):
        src = pl.ds(pl.multiple_of(e0 + j * CHUNK, 8), CHUNK)
        pltpu.async_copy(col_hbm.at[src], colv.at[ib], isem[ib])
        pltpu.async_copy(row_hbm.at[src], rowv.at[ib], isem[ib])
        pltpu.async_copy(val_hbm.at[src], valv.at[ib], isem[ib])

    def wait_idx(j, ib):
        src = pl.ds(pl.multiple_of(e0 + j * CHUNK, 8), CHUNK)
        pltpu.make_async_copy(col_hbm.at[src], colv.at[ib], isem[ib]).wait()
        pltpu.make_async_copy(row_hbm.at[src], rowv.at[ib], isem[ib]).wait()
        pltpu.make_async_copy(val_hbm.at[src], valv.at[ib], isem[ib]).wait()

    def start_gather(ib, b):
        pltpu.async_copy(x_hbm.at[colv.at[ib]], rows[b], gsem[b])

    def wait_gather(ib, b):
        pltpu.make_async_copy(x_hbm.at[colv.at[ib]], rows[b], gsem[b]).wait()

    def scale(ib, b):
        def group_body(g, c2):
            vals16 = valv[ib, pl.ds(g * 16, 16)]
            for t in range(16):
                e = g * 16 + t
                v = vals16[t]
                for db in range(D // 16):
                    sl = pl.ds(db * 16, 16)
                    rows[b][e, sl] = rows[b][e, sl] * v
            return c2

        lax.fori_loop(0, CHUNK // 16, group_body, 0)

    def scatter(ib, b):
        pltpu.sync_copy(rows[b], acc.at[rowv.at[ib]], add=True)

    start_idx(0, 0)
    start_idx(1, 1)
    wait_idx(0, 0)
    start_gather(0, 0)
    wait_idx(1, 1)
    start_gather(1, 1)
    start_idx(2, 2)

    def chunk_body(p, carry):
        for b in range(3):
            j = 3 * p + b
            b2 = (b + 2) % 3

            @pl.when(j < NCHUNK)
            def _():
                @pl.when(j + 2 < NCHUNK)
                def _():
                    wait_idx(j + 2, b2)
                    start_gather(b2, b2)

                wait_gather(b, b)
                scale(b, b)
                scatter(b, b)

                @pl.when(j + 3 < NCHUNK)
                def _():
                    start_idx(j + 3, b)
        return carry

    lax.fori_loop(0, (NCHUNK + 2) // 3, chunk_body, 0)
    plsc.subcore_barrier()

    for kk in range(RITER):
        rc = sid + kk * NS

        @pl.when(rc < NRCHUNK)
        def _():
            r0 = rc * OBUF_ROWS
            pltpu.sync_copy(acc.at[pl.ds(r0, OBUF_ROWS)], obuf)
            pltpu.sync_copy(obuf, out_hbm.at[cid, pl.ds(r0, OBUF_ROWS)])


_spmm_sc = functools.partial(
    pl.kernel,
    mesh=plsc.VectorSubcoreMesh(core_axis_name="c", subcore_axis_name="s"),
    out_type=jax.ShapeDtypeStruct((NC, N, D), jnp.float32),
    scratch_types=[
        pltpu.VMEM((3, CHUNK), jnp.int32),
        pltpu.VMEM((3, CHUNK), jnp.int32),
        pltpu.VMEM((3, CHUNK), jnp.float32),
        pltpu.VMEM((CHUNK, D), jnp.float32),
        pltpu.VMEM((CHUNK, D), jnp.float32),
        pltpu.VMEM((CHUNK, D), jnp.float32),
        pltpu.VMEM_SHARED((N, D), jnp.float32),
        pltpu.SemaphoreType.DMA,
        pltpu.SemaphoreType.DMA,
        pltpu.SemaphoreType.DMA,
        pltpu.SemaphoreType.DMA,
        pltpu.SemaphoreType.DMA,
        pltpu.SemaphoreType.DMA,
    ],
)(_spmm_body)


BLK = 1000


def _dense_body(part_ref, ego_ref, w1_ref, b1_ref, w2_ref, b2_ref,
                egon_ref, norm_ref):
    side = part_ref[0] + part_ref[1]
    ego = ego_ref[...]
    s1 = lax.dot_general(side, w1_ref[...], (((1,), (1,)), ((), ())),
                         preferred_element_type=jnp.float32) + b1_ref[...]
    s1 = jnp.where(s1 >= 0, s1, 0.01 * s1)
    s2 = lax.dot_general(ego * side, w2_ref[...], (((1,), (1,)), ((), ())),
                         preferred_element_type=jnp.float32) + b2_ref[...]
    s2 = jnp.where(s2 >= 0, s2, 0.01 * s2)
    e = s1 + s2
    egon_ref[...] = e
    nrm = jnp.sqrt(jnp.sum(e * e, axis=1, keepdims=True))
    norm_ref[...] = e / jnp.maximum(nrm, 1e-12)


_dense_tc = pl.pallas_call(
    _dense_body,
    grid=(N // BLK,),
    in_specs=[
        pl.BlockSpec((NC, BLK, D), lambda i: (0, i, 0)),
        pl.BlockSpec((BLK, D), lambda i: (i, 0)),
        pl.BlockSpec((D, D), lambda i: (0, 0)),
        pl.BlockSpec((1, D), lambda i: (0, 0)),
        pl.BlockSpec((D, D), lambda i: (0, 0)),
        pl.BlockSpec((1, D), lambda i: (0, 0)),
    ],
    out_specs=[
        pl.BlockSpec((BLK, D), lambda i: (i, 0)),
        pl.BlockSpec((BLK, D), lambda i: (i, 0)),
    ],
    out_shape=[
        jax.ShapeDtypeStruct((N, D), jnp.float32),
        jax.ShapeDtypeStruct((N, D), jnp.float32),
    ],
)


def kernel(author_embedding, paper_embedding, adj_values, W1, b1, W2, b2,
           edge_index):
    ego = jnp.concatenate([author_embedding, paper_embedding], axis=0)
    row = edge_index[0]
    col = edge_index[1]
    outs = [ego]
    for k in range(L):
        part = _spmm_sc(ego, col, row, adj_values)
        ego, nrm = _dense_tc(part, ego, W1[k], b1[k].reshape(1, D),
                             W2[k], b2[k].reshape(1, D))
        outs.append(nrm)
    all_emb = jnp.concatenate(outs, axis=1)
    return (all_emb[:N_AUTHORS], all_emb[N_AUTHORS:])


# speedup vs baseline: 1.2477x; 1.2477x over previous
import functools

import jax
import jax.numpy as jnp
from jax import lax
from jax.experimental import pallas as pl
from jax.experimental.pallas import tpu as pltpu
from jax.experimental.pallas import tpu_sc as plsc

N_AUTHORS = 5000
N_PAPERS = 5000
N = N_AUTHORS + N_PAPERS
E = 320000
D = 128
L = 3

NC = 2
NS = 16
NW = NC * NS
EPW = E // NW
CHUNK = 80
NCHUNK = EPW // CHUNK

OBUF_ROWS = 80
NRCHUNK = N // OBUF_ROWS
RITER = -(-NRCHUNK // NS)


def _spmm_body(x_hbm, col_hbm, row_hbm, val_hbm, out_hbm,
               colv, rowv, valv, rowsc, rows0, rows1, rows2, acc,
               gs0, gs1, gs2, ss0, ss1, ss2, is0, is1, is2):
    cid = lax.axis_index("c")
    sid = lax.axis_index("s")
    wid = cid * NS + sid
    rows = (rows0, rows1, rows2)
    gsem = (gs0, gs1, gs2)
    ssem = (ss0, ss1, ss2)
    isem = (is0, is1, is2)
    obuf = rows0.at[pl.ds(0, OBUF_ROWS)]

    zv = jnp.zeros((16,), jnp.float32)

    def zero_row(i, carry):
        for db in range(D // 16):
            rows0[i, pl.ds(db * 16, 16)] = zv
        return carry

    lax.fori_loop(0, OBUF_ROWS, zero_row, 0)
    for kk in range(RITER):
        rc = sid + kk * NS

        @pl.when(rc < NRCHUNK)
        def _():
            pltpu.sync_copy(obuf, acc.at[pl.ds(rc * OBUF_ROWS, OBUF_ROWS)])
    plsc.subcore_barrier()

    e0 = wid * EPW

    def start_idx(j, ib):
        src = pl.ds(pl.multiple_of(e0 + j * CHUNK, 8), CHUNK)
        pltpu.async_copy(col_hbm.at[src], colv.at[ib], isem[ib])
        pltpu.async_copy(row_hbm.at[src], rowv.at[ib], isem[ib])
        pltpu.async_copy(val_hbm.at[src], valv.at[ib], isem[ib])

    def wait_idx(j, ib):
        src = pl.ds(pl.multiple_of(e0 + j * CHUNK, 8), CHUNK)
        pltpu.make_async_copy(col_hbm.at[src], colv.at[ib], isem[ib]).wait()
        pltpu.make_async_copy(row_hbm.at[src], rowv.at[ib], isem[ib]).wait()
        pltpu.make_async_copy(val_hbm.at[src], valv.at[ib], isem[ib]).wait()

    def start_gather(ib, b):
        pltpu.async_copy(x_hbm.at[colv.at[ib]], rows[b], gsem[b])

    def wait_gather(ib, b):
        pltpu.make_async_copy(x_hbm.at[colv.at[ib]], rows[b], gsem[b]).wait()

    def scale(ib, b):
        def group_body(g, c2):
            vals16 = valv[ib, pl.ds(g * 16, 16)]
            for t in range(16):
                e = g * 16 + t
                v = vals16[t]
                for db in range(D // 16):
                    sl = pl.ds(db * 16, 16)
                    rows[b][e, sl] = rows[b][e, sl] * v
            return c2

        lax.fori_loop(0, CHUNK // 16, group_body, 0)

    def stage_rows_idx(ib, b):
        def cp(g, c2):
            rowsc[b, pl.ds(g * 16, 16)] = rowv[ib, pl.ds(g * 16, 16)]
            return c2

        lax.fori_loop(0, CHUNK // 16, cp, 0)

    def start_scatter(b):
        pltpu.async_copy(rows[b], acc.at[rowsc.at[b]], ssem[b], add=True)

    def wait_scatter(b):
        pltpu.make_async_copy(rows[b], acc.at[rowsc.at[b]], ssem[b]).wait()

    start_idx(0, 0)
    start_idx(1, 1)
    wait_idx(0, 0)
    start_gather(0, 0)
    wait_idx(1, 1)
    start_gather(1, 1)
    start_idx(2, 2)

    def chunk_body(p, carry):
        for b in range(3):
            j = 3 * p + b
            b2 = (b + 2) % 3

            @pl.when(j < NCHUNK)
            def _():
                @pl.when(j + 2 < NCHUNK)
                def _():
                    wait_idx(j + 2, b2)

                    @pl.when(j >= 1)
                    def _():
                        wait_scatter(b2)
                    start_gather(b2, b2)

                wait_gather(b, b)
                scale(b, b)
                stage_rows_idx(b, b)
                start_scatter(b)

                @pl.when(j + 3 < NCHUNK)
                def _():
                    start_idx(j + 3, b)
        return carry

    lax.fori_loop(0, (NCHUNK + 2) // 3, chunk_body, 0)
    for j in (NCHUNK - 3, NCHUNK - 2, NCHUNK - 1):
        wait_scatter(j % 3)
    plsc.subcore_barrier()

    for kk in range(RITER):
        rc = sid + kk * NS

        @pl.when(rc < NRCHUNK)
        def _():
            r0 = rc * OBUF_ROWS
            pltpu.sync_copy(acc.at[pl.ds(r0, OBUF_ROWS)], obuf)
            pltpu.sync_copy(obuf, out_hbm.at[cid, pl.ds(r0, OBUF_ROWS)])


_spmm_sc = functools.partial(
    pl.kernel,
    mesh=plsc.VectorSubcoreMesh(core_axis_name="c", subcore_axis_name="s"),
    out_type=jax.ShapeDtypeStruct((NC, N, D), jnp.float32),
    scratch_types=[
        pltpu.VMEM((3, CHUNK), jnp.int32),
        pltpu.VMEM((3, CHUNK), jnp.int32),
        pltpu.VMEM((3, CHUNK), jnp.float32),
        pltpu.VMEM((3, CHUNK), jnp.int32),
        pltpu.VMEM((CHUNK, D), jnp.float32),
        pltpu.VMEM((CHUNK, D), jnp.float32),
        pltpu.VMEM((CHUNK, D), jnp.float32),
        pltpu.VMEM_SHARED((N, D), jnp.float32),
        pltpu.SemaphoreType.DMA,
        pltpu.SemaphoreType.DMA,
        pltpu.SemaphoreType.DMA,
        pltpu.SemaphoreType.DMA,
        pltpu.SemaphoreType.DMA,
        pltpu.SemaphoreType.DMA,
        pltpu.SemaphoreType.DMA,
        pltpu.SemaphoreType.DMA,
        pltpu.SemaphoreType.DMA,
    ],
)(_spmm_body)


BLK = 1000


def _dense_body(part_ref, ego_ref, w1_ref, b1_ref, w2_ref, b2_ref,
                egon_ref, norm_ref):
    side = part_ref[0] + part_ref[1]
    ego = ego_ref[...]
    s1 = lax.dot_general(side, w1_ref[...], (((1,), (1,)), ((), ())),
                         preferred_element_type=jnp.float32) + b1_ref[...]
    s1 = jnp.where(s1 >= 0, s1, 0.01 * s1)
    s2 = lax.dot_general(ego * side, w2_ref[...], (((1,), (1,)), ((), ())),
                         preferred_element_type=jnp.float32) + b2_ref[...]
    s2 = jnp.where(s2 >= 0, s2, 0.01 * s2)
    e = s1 + s2
    egon_ref[...] = e
    nrm = jnp.sqrt(jnp.sum(e * e, axis=1, keepdims=True))
    norm_ref[...] = e / jnp.maximum(nrm, 1e-12)


_dense_tc = pl.pallas_call(
    _dense_body,
    grid=(N // BLK,),
    in_specs=[
        pl.BlockSpec((NC, BLK, D), lambda i: (0, i, 0)),
        pl.BlockSpec((BLK, D), lambda i: (i, 0)),
        pl.BlockSpec((D, D), lambda i: (0, 0)),
        pl.BlockSpec((1, D), lambda i: (0, 0)),
        pl.BlockSpec((D, D), lambda i: (0, 0)),
        pl.BlockSpec((1, D), lambda i: (0, 0)),
    ],
    out_specs=[
        pl.BlockSpec((BLK, D), lambda i: (i, 0)),
        pl.BlockSpec((BLK, D), lambda i: (i, 0)),
    ],
    out_shape=[
        jax.ShapeDtypeStruct((N, D), jnp.float32),
        jax.ShapeDtypeStruct((N, D), jnp.float32),
    ],
)


def kernel(author_embedding, paper_embedding, adj_values, W1, b1, W2, b2,
           edge_index):
    ego = jnp.concatenate([author_embedding, paper_embedding], axis=0)
    row = edge_index[0]
    col = edge_index[1]
    outs = [ego]
    for k in range(L):
        part = _spmm_sc(ego, col, row, adj_values)
        ego, nrm = _dense_tc(part, ego, W1[k], b1[k].reshape(1, D),
                             W2[k], b2[k].reshape(1, D))
        outs.append(nrm)
    all_emb = jnp.concatenate(outs, axis=1)
    return (all_emb[:N_AUTHORS], all_emb[N_AUTHORS:])


# speedup vs baseline: 1.3314x; 1.0671x over previous
import functools

import jax
import jax.numpy as jnp
from jax import lax
from jax.experimental import pallas as pl
from jax.experimental.pallas import tpu as pltpu
from jax.experimental.pallas import tpu_sc as plsc

N_AUTHORS = 5000
N_PAPERS = 5000
N = N_AUTHORS + N_PAPERS
E = 320000
D = 128
L = 3

NC = 2
NS = 16
NW = NC * NS
EPW = E // NW
CHUNK = 80
NCHUNK = EPW // CHUNK
PW = 256
SUPC = 16
NSUPER = -(-NCHUNK // SUPC)

OBUF_ROWS = 80
NRCHUNK = N // OBUF_ROWS
RITER = -(-NRCHUNK // NS)


def _spmm_body(x_hbm, p_hbm, out_hbm,
               sup, rowsc, rows0, rows1, rows2, acc,
               gs0, gs1, gs2, ss0, ss1, ss2, is0, is1):
    cid = lax.axis_index("c")
    sid = lax.axis_index("s")
    wid = cid * NS + sid
    rows = (rows0, rows1, rows2)
    gsem = (gs0, gs1, gs2)
    ssem = (ss0, ss1, ss2)
    isem = (is0, is1)
    obuf = rows0.at[pl.ds(0, OBUF_ROWS)]

    zv = jnp.zeros((16,), jnp.float32)

    def zero_row(i, carry):
        for db in range(D // 16):
            rows0[i, pl.ds(db * 16, 16)] = zv
        return carry

    lax.fori_loop(0, OBUF_ROWS, zero_row, 0)
    for kk in range(RITER):
        rc = sid + kk * NS

        @pl.when(rc < NRCHUNK)
        def _():
            pltpu.sync_copy(obuf, acc.at[pl.ds(rc * OBUF_ROWS, OBUF_ROWS)])
    plsc.subcore_barrier()

    c0 = wid * NCHUNK

    iota16 = lax.iota(jnp.int32, 16)

    def _start_super_sb(sp, sb):
        ivec = iota16 + (c0 + sp * SUPC)
        pltpu.async_copy(p_hbm.at[ivec], sup.at[sb], isem[sb])

    def start_super(sp):
        if isinstance(sp, int):
            _start_super_sb(sp, sp % 2)
        else:
            for sb in range(2):
                @pl.when(sp % 2 == sb)
                def _(sb=sb):
                    _start_super_sb(sp, sb)

    def wait_super(sp):
        if isinstance(sp, int):
            sbs = [sp % 2]
            pltpu.make_async_copy(p_hbm.at[iota16], sup.at[sbs[0]],
                                  isem[sbs[0]]).wait()
        else:
            for sb in range(2):
                @pl.when(sp % 2 == sb)
                def _(sb=sb):
                    pltpu.make_async_copy(p_hbm.at[iota16], sup.at[sb],
                                          isem[sb]).wait()

    def start_gather(j, b):
        sb = (j // SUPC) % 2
        jj = j % SUPC
        pltpu.async_copy(x_hbm.at[sup.at[sb, jj, pl.ds(0, CHUNK)]],
                         rows[b], gsem[b])

    def wait_gather(b):
        pltpu.make_async_copy(x_hbm.at[pl.ds(0, CHUNK)], rows[b],
                              gsem[b]).wait()

    def scale(j, b):
        sb = (j // SUPC) % 2
        jj = j % SUPC

        def group_body(g, c2):
            vals16 = lax.bitcast_convert_type(
                sup[sb, jj, pl.ds(2 * CHUNK + g * 16, 16)], jnp.float32)
            for t in range(16):
                e = g * 16 + t
                v = vals16[t]
                for db in range(D // 16):
                    sl = pl.ds(db * 16, 16)
                    rows[b][e, sl] = rows[b][e, sl] * v
            return c2

        lax.fori_loop(0, CHUNK // 16, group_body, 0)

    def stage_rows_idx(j, b):
        sb = (j // SUPC) % 2
        jj = j % SUPC

        def cp(g, c2):
            rowsc[b, pl.ds(g * 16, 16)] = sup[sb, jj,
                                              pl.ds(CHUNK + g * 16, 16)]
            return c2

        lax.fori_loop(0, CHUNK // 16, cp, 0)

    def start_scatter(b):
        pltpu.async_copy(rows[b], acc.at[rowsc.at[b]], ssem[b], add=True)

    def wait_scatter(b):
        pltpu.make_async_copy(rows[b], acc.at[pl.ds(0, CHUNK)],
                              ssem[b]).wait()

    start_super(0)
    wait_super(0)
    start_gather(0, 0)
    start_gather(1, 1)
    start_super(1)

    def chunk_body(p, carry):
        for b in range(3):
            j = 3 * p + b
            b2 = (b + 2) % 3

            @pl.when(j < NCHUNK)
            def _():
                @pl.when(j + 2 < NCHUNK)
                def _():
                    @pl.when((j + 2) % SUPC == 0)
                    def _():
                        wait_super((j + 2) // SUPC)

                    @pl.when(j >= 1)
                    def _():
                        wait_scatter(b2)
                    start_gather(j + 2, b2)

                wait_gather(b)
                scale(j, b)
                stage_rows_idx(j, b)
                start_scatter(b)

                @pl.when((j % SUPC == 2) & (j >= SUPC + 2)
                         & (j // SUPC + 1 < NSUPER))
                def _():
                    start_super(j // SUPC + 1)
        return carry

    lax.fori_loop(0, (NCHUNK + 2) // 3, chunk_body, 0)
    for j in (NCHUNK - 3, NCHUNK - 2, NCHUNK - 1):
        wait_scatter(j % 3)
    plsc.subcore_barrier()

    for kk in range(RITER):
        rc = sid + kk * NS

        @pl.when(rc < NRCHUNK)
        def _():
            r0 = rc * OBUF_ROWS
            pltpu.sync_copy(acc.at[pl.ds(r0, OBUF_ROWS)], obuf)
            pltpu.sync_copy(obuf, out_hbm.at[cid, pl.ds(r0, OBUF_ROWS)])


_spmm_sc = functools.partial(
    pl.kernel,
    mesh=plsc.VectorSubcoreMesh(core_axis_name="c", subcore_axis_name="s"),
    out_type=jax.ShapeDtypeStruct((NC, N, D), jnp.float32),
    scratch_types=[
        pltpu.VMEM((2, SUPC, PW), jnp.int32),
        pltpu.VMEM((3, CHUNK), jnp.int32),
        pltpu.VMEM((CHUNK, D), jnp.float32),
        pltpu.VMEM((CHUNK, D), jnp.float32),
        pltpu.VMEM((CHUNK, D), jnp.float32),
        pltpu.VMEM_SHARED((N, D), jnp.float32),
        pltpu.SemaphoreType.DMA,
        pltpu.SemaphoreType.DMA,
        pltpu.SemaphoreType.DMA,
        pltpu.SemaphoreType.DMA,
        pltpu.SemaphoreType.DMA,
        pltpu.SemaphoreType.DMA,
        pltpu.SemaphoreType.DMA,
        pltpu.SemaphoreType.DMA,
    ],
)(_spmm_body)


BLK = 1000


def _dense_body(part_ref, ego_ref, w1_ref, b1_ref, w2_ref, b2_ref,
                egon_ref, norm_ref):
    side = part_ref[0] + part_ref[1]
    ego = ego_ref[...]
    s1 = lax.dot_general(side, w1_ref[...], (((1,), (1,)), ((), ())),
                         preferred_element_type=jnp.float32) + b1_ref[...]
    s1 = jnp.where(s1 >= 0, s1, 0.01 * s1)
    s2 = lax.dot_general(ego * side, w2_ref[...], (((1,), (1,)), ((), ())),
                         preferred_element_type=jnp.float32) + b2_ref[...]
    s2 = jnp.where(s2 >= 0, s2, 0.01 * s2)
    e = s1 + s2
    egon_ref[...] = e
    nrm = jnp.sqrt(jnp.sum(e * e, axis=1, keepdims=True))
    norm_ref[...] = e / jnp.maximum(nrm, 1e-12)


_dense_tc = pl.pallas_call(
    _dense_body,
    grid=(N // BLK,),
    in_specs=[
        pl.BlockSpec((NC, BLK, D), lambda i: (0, i, 0)),
        pl.BlockSpec((BLK, D), lambda i: (i, 0)),
        pl.BlockSpec((D, D), lambda i: (0, 0)),
        pl.BlockSpec((1, D), lambda i: (0, 0)),
        pl.BlockSpec((D, D), lambda i: (0, 0)),
        pl.BlockSpec((1, D), lambda i: (0, 0)),
    ],
    out_specs=[
        pl.BlockSpec((BLK, D), lambda i: (i, 0)),
        pl.BlockSpec((BLK, D), lambda i: (i, 0)),
    ],
    out_shape=[
        jax.ShapeDtypeStruct((N, D), jnp.float32),
        jax.ShapeDtypeStruct((N, D), jnp.float32),
    ],
)


def kernel(author_embedding, paper_embedding, adj_values, W1, b1, W2, b2,
           edge_index):
    ego = jnp.concatenate([author_embedding, paper_embedding], axis=0)
    col2 = edge_index[1].reshape(NW * NCHUNK, CHUNK)
    row2 = edge_index[0].reshape(NW * NCHUNK, CHUNK)
    val2 = jax.lax.bitcast_convert_type(
        adj_values.reshape(NW * NCHUNK, CHUNK), jnp.int32)
    zpad = jnp.zeros((NW * NCHUNK, PW - 3 * CHUNK), jnp.int32)
    packed = jnp.concatenate([col2, row2, val2, zpad], axis=1)
    packed = jnp.concatenate(
        [packed, jnp.zeros((SUPC, PW), jnp.int32)], axis=0)
    outs = [ego]
    for k in range(L):
        part = _spmm_sc(ego, packed)
        ego, nrm = _dense_tc(part, ego, W1[k], b1[k].reshape(1, D),
                             W2[k], b2[k].reshape(1, D))
        outs.append(nrm)
    all_emb = jnp.concatenate(outs, axis=1)
    return (all_emb[:N_AUTHORS], all_emb[N_AUTHORS:])


# speedup vs baseline: 1.3606x; 1.0220x over previous
import functools

import jax
import jax.numpy as jnp
from jax import lax
from jax.experimental import pallas as pl
from jax.experimental.pallas import tpu as pltpu
from jax.experimental.pallas import tpu_sc as plsc

N_AUTHORS = 5000
N_PAPERS = 5000
N = N_AUTHORS + N_PAPERS
E = 320000
D = 128
L = 3

NC = 2
NS = 16
NW = NC * NS
EPW = E // NW
CHUNK = 80
NCHUNK = EPW // CHUNK
PW = 256
SUPC = 16
NSUPER = -(-NCHUNK // SUPC)

OBUF_ROWS = 80
NRCHUNK = N // OBUF_ROWS
RITER = -(-NRCHUNK // NS)


def _spmm_body(x_hbm, p_hbm, out_hbm,
               sup, rowsc, rows0, rows1, rows2, acc,
               gs0, gs1, gs2, ss0, ss1, ss2, is0, is1):
    cid = lax.axis_index("c")
    sid = lax.axis_index("s")
    wid = cid * NS + sid
    rows = (rows0, rows1, rows2)
    gsem = (gs0, gs1, gs2)
    ssem = (ss0, ss1, ss2)
    isem = (is0, is1)
    obuf = rows0.at[pl.ds(0, OBUF_ROWS)]
    obuf1 = rows1.at[pl.ds(0, OBUF_ROWS)]

    def zero_acc():
        zv = jnp.zeros((16,), jnp.float32)

        def zero_row(i, carry):
            for db in range(D // 16):
                rows0[i, pl.ds(db * 16, 16)] = zv
            return carry

        lax.fori_loop(0, OBUF_ROWS, zero_row, 0)
        for kk in range(RITER):
            rc = sid + kk * NS

            @pl.when(rc < NRCHUNK)
            def _():
                pltpu.sync_copy(obuf,
                                acc.at[pl.ds(rc * OBUF_ROWS, OBUF_ROWS)])
        plsc.subcore_barrier()

    c0 = wid * NCHUNK

    iota16 = lax.iota(jnp.int32, 16)

    def _start_super_sb(sp, sb):
        ivec = iota16 + (c0 + sp * SUPC)
        pltpu.async_copy(p_hbm.at[ivec], sup.at[sb], isem[sb])

    def start_super(sp):
        if isinstance(sp, int):
            _start_super_sb(sp, sp % 2)
        else:
            for sb in range(2):
                @pl.when(sp % 2 == sb)
                def _(sb=sb):
                    _start_super_sb(sp, sb)

    def wait_super(sp):
        if isinstance(sp, int):
            sbs = [sp % 2]
            pltpu.make_async_copy(p_hbm.at[iota16], sup.at[sbs[0]],
                                  isem[sbs[0]]).wait()
        else:
            for sb in range(2):
                @pl.when(sp % 2 == sb)
                def _(sb=sb):
                    pltpu.make_async_copy(p_hbm.at[iota16], sup.at[sb],
                                          isem[sb]).wait()

    def start_gather(j, b):
        sb = (j // SUPC) % 2
        jj = j % SUPC
        pltpu.async_copy(x_hbm.at[sup.at[sb, jj, pl.ds(0, CHUNK)]],
                         rows[b], gsem[b])

    def wait_gather(b):
        pltpu.make_async_copy(x_hbm.at[pl.ds(0, CHUNK)], rows[b],
                              gsem[b]).wait()

    def scale(j, b):
        sb = (j // SUPC) % 2
        jj = j % SUPC

        def group_body(g, c2):
            sl16 = pl.ds(g * 16, 16)
            vals16 = lax.bitcast_convert_type(
                sup[sb, jj, pl.ds(2 * CHUNK + g * 16, 16)], jnp.float32)
            rowsc[b, sl16] = sup[sb, jj, pl.ds(CHUNK + g * 16, 16)]
            for t in range(16):
                e = g * 16 + t
                v = vals16[t]
                for db in range(D // 16):
                    sl = pl.ds(db * 16, 16)
                    rows[b][e, sl] = rows[b][e, sl] * v
            return c2

        lax.fori_loop(0, CHUNK // 16, group_body, 0)

    def start_scatter(b):
        pltpu.async_copy(rows[b], acc.at[rowsc.at[b]], ssem[b], add=True)

    def wait_scatter(b):
        pltpu.make_async_copy(rows[b], acc.at[pl.ds(0, CHUNK)],
                              ssem[b]).wait()

    start_super(0)
    zero_acc()
    wait_super(0)
    start_gather(0, 0)
    start_gather(1, 1)
    start_super(1)

    def chunk_body(p, carry):
        for b in range(3):
            j = 3 * p + b
            b2 = (b + 2) % 3

            @pl.when(j < NCHUNK)
            def _():
                @pl.when(j + 2 < NCHUNK)
                def _():
                    @pl.when((j + 2) % SUPC == 0)
                    def _():
                        wait_super((j + 2) // SUPC)

                    @pl.when(j >= 1)
                    def _():
                        wait_scatter(b2)
                    start_gather(j + 2, b2)

                wait_gather(b)
                scale(j, b)
                start_scatter(b)

                @pl.when((j % SUPC == 2) & (j >= SUPC + 2)
                         & (j // SUPC + 1 < NSUPER))
                def _():
                    start_super(j // SUPC + 1)
        return carry

    lax.fori_loop(0, (NCHUNK + 2) // 3, chunk_body, 0)
    for j in (NCHUNK - 3, NCHUNK - 2, NCHUNK - 1):
        wait_scatter(j % 3)
    plsc.subcore_barrier()

    obufs = (obuf, obuf1)

    def dump_wait(kk):
        ob = obufs[kk % 2]
        sm = gsem[kk % 2]

        @pl.when(sid + kk * NS < NRCHUNK)
        def _():
            pltpu.make_async_copy(
                ob, out_hbm.at[cid, pl.ds(0, OBUF_ROWS)], sm).wait()

    for kk in range(RITER):
        rc = sid + kk * NS
        ob = obufs[kk % 2]
        sm = gsem[kk % 2]
        if kk >= 2:
            dump_wait(kk - 2)

        @pl.when(rc < NRCHUNK)
        def _(rc=rc, ob=ob, sm=sm):
            r0 = rc * OBUF_ROWS
            pltpu.sync_copy(acc.at[pl.ds(r0, OBUF_ROWS)], ob)
            pltpu.async_copy(ob, out_hbm.at[cid, pl.ds(r0, OBUF_ROWS)], sm)
    dump_wait(RITER - 2)
    dump_wait(RITER - 1)


_spmm_sc = functools.partial(
    pl.kernel,
    mesh=plsc.VectorSubcoreMesh(core_axis_name="c", subcore_axis_name="s"),
    out_type=jax.ShapeDtypeStruct((NC, N, D), jnp.float32),
    scratch_types=[
        pltpu.VMEM((2, SUPC, PW), jnp.int32),
        pltpu.VMEM((3, CHUNK), jnp.int32),
        pltpu.VMEM((CHUNK, D), jnp.float32),
        pltpu.VMEM((CHUNK, D), jnp.float32),
        pltpu.VMEM((CHUNK, D), jnp.float32),
        pltpu.VMEM_SHARED((N, D), jnp.float32),
        pltpu.SemaphoreType.DMA,
        pltpu.SemaphoreType.DMA,
        pltpu.SemaphoreType.DMA,
        pltpu.SemaphoreType.DMA,
        pltpu.SemaphoreType.DMA,
        pltpu.SemaphoreType.DMA,
        pltpu.SemaphoreType.DMA,
        pltpu.SemaphoreType.DMA,
    ],
)(_spmm_body)


BLK = 1000


def _dense_body(part_ref, ego_ref, w1_ref, b1_ref, w2_ref, b2_ref,
                egon_ref, norm_ref):
    side = part_ref[0] + part_ref[1]
    ego = ego_ref[...]
    s1 = lax.dot_general(side, w1_ref[...], (((1,), (1,)), ((), ())),
                         preferred_element_type=jnp.float32) + b1_ref[...]
    s1 = jnp.where(s1 >= 0, s1, 0.01 * s1)
    s2 = lax.dot_general(ego * side, w2_ref[...], (((1,), (1,)), ((), ())),
                         preferred_element_type=jnp.float32) + b2_ref[...]
    s2 = jnp.where(s2 >= 0, s2, 0.01 * s2)
    e = s1 + s2
    egon_ref[...] = e
    nrm = jnp.sqrt(jnp.sum(e * e, axis=1, keepdims=True))
    norm_ref[...] = e / jnp.maximum(nrm, 1e-12)


_dense_tc = pl.pallas_call(
    _dense_body,
    grid=(N // BLK,),
    in_specs=[
        pl.BlockSpec((NC, BLK, D), lambda i: (0, i, 0)),
        pl.BlockSpec((BLK, D), lambda i: (i, 0)),
        pl.BlockSpec((D, D), lambda i: (0, 0)),
        pl.BlockSpec((1, D), lambda i: (0, 0)),
        pl.BlockSpec((D, D), lambda i: (0, 0)),
        pl.BlockSpec((1, D), lambda i: (0, 0)),
    ],
    out_specs=[
        pl.BlockSpec((BLK, D), lambda i: (i, 0)),
        pl.BlockSpec((BLK, D), lambda i: (i, 0)),
    ],
    out_shape=[
        jax.ShapeDtypeStruct((N, D), jnp.float32),
        jax.ShapeDtypeStruct((N, D), jnp.float32),
    ],
)


def kernel(author_embedding, paper_embedding, adj_values, W1, b1, W2, b2,
           edge_index):
    ego = jnp.concatenate([author_embedding, paper_embedding], axis=0)
    col2 = edge_index[1].reshape(NW * NCHUNK, CHUNK)
    row2 = edge_index[0].reshape(NW * NCHUNK, CHUNK)
    val2 = jax.lax.bitcast_convert_type(
        adj_values.reshape(NW * NCHUNK, CHUNK), jnp.int32)
    zpad = jnp.zeros((NW * NCHUNK, PW - 3 * CHUNK), jnp.int32)
    packed = jnp.concatenate([col2, row2, val2, zpad], axis=1)
    packed = jnp.concatenate(
        [packed, jnp.zeros((SUPC, PW), jnp.int32)], axis=0)
    outs = [ego]
    for k in range(L):
        part = _spmm_sc(ego, packed)
        ego, nrm = _dense_tc(part, ego, W1[k], b1[k].reshape(1, D),
                             W2[k], b2[k].reshape(1, D))
        outs.append(nrm)
    all_emb = jnp.concatenate(outs, axis=1)
    return (all_emb[:N_AUTHORS], all_emb[N_AUTHORS:])
